# baseline (speedup 1.0000x reference)
import functools

import jax
import jax.numpy as jnp
from jax import lax
from jax.experimental import pallas as pl
from jax.experimental.pallas import tpu as pltpu
from jax.experimental.pallas import tpu_sc as plsc

N_NODES = 10000
N_EDGES = 320000
D = 128

NC = 2
NS = 16
NW = NC * NS
CH = 128
BL = 8
NBLK = 10
CPT = NBLK * BL
EDGES_PER_TILE = CPT * CH
REAL_PER_TILE = N_EDGES // NW
PAD_PER_TILE = EDGES_PER_TILE - REAL_PER_TILE
N_PAD = 10112
TRASH = N_PAD - N_NODES
ROWS_PER_TILE = N_PAD // NS


@functools.cache
def _make_agg_kernel():
    mesh = plsc.VectorSubcoreMesh(core_axis_name="c", subcore_axis_name="s")

    @functools.partial(
        pl.kernel,
        mesh=mesh,
        out_type=jax.ShapeDtypeStruct((NC, N_PAD, D), jnp.float32),
        scratch_types=[
            pltpu.VMEM((CPT, CH), jnp.int32),
            pltpu.VMEM((CPT, CH), jnp.int32),
            pltpu.VMEM((CH, D), jnp.float32),
            pltpu.VMEM_SHARED((N_PAD, D), jnp.float32),
            pltpu.SemaphoreType.DMA,
        ],
    )
    def agg(h_hbm, src_hbm, dst_hbm, out_hbm, src_v, dst_v, rows0, acc, sem0):
        c = lax.axis_index("c")
        s = lax.axis_index("s")

        pltpu.sync_copy(src_hbm.at[c, s], src_v)
        pltpu.sync_copy(dst_hbm.at[c, s], dst_v)

        def zrow(r, carry):
            for k in range(D // 16):
                rows0[r, pl.ds(k * 16, 16)] = jnp.zeros((16,), jnp.float32)
            return carry

        lax.fori_loop(0, CH, zrow, 0)
        base = s * ROWS_PER_TILE
        full = ROWS_PER_TILE // CH
        rem = ROWS_PER_TILE - full * CH
        for k in range(full):
            pltpu.sync_copy(rows0, acc.at[pl.ds(base + k * CH, CH)])
        if rem:
            pltpu.sync_copy(rows0.at[pl.ds(0, rem)],
                            acc.at[pl.ds(base + full * CH, rem)])
        plsc.subcore_barrier()

        def chunk(j, carry):
            pltpu.async_copy(h_hbm.at[src_v.at[j]], rows0, sem0).wait()
            pltpu.sync_copy(rows0, acc.at[dst_v.at[j]], add=True)
            return carry

        lax.fori_loop(0, CPT, chunk, 0)
        plsc.subcore_barrier()

        pltpu.sync_copy(acc.at[pl.ds(base, ROWS_PER_TILE)],
                        out_hbm.at[c, pl.ds(base, ROWS_PER_TILE)])

    return agg


_ROW_BLK = 1000


def _mlp1_body(h_ref, p_ref, w1_ref, b1_ref, w2_ref, b2_ref, o_ref):
    z = h_ref[...] + p_ref[0] + p_ref[1]
    a = jnp.dot(z, w1_ref[...], preferred_element_type=jnp.float32) + b1_ref[...]
    a = jnp.maximum(a, 0.0)
    z2 = jnp.dot(a, w2_ref[...], preferred_element_type=jnp.float32) + b2_ref[...]
    o_ref[...] = jnp.maximum(z2, 0.0)


def _mlp2_body(h_ref, p_ref, w1_ref, b1_ref, w2_ref, b2_ref, wr_ref, br_ref,
               o_ref):
    z = h_ref[...] + p_ref[0] + p_ref[1]
    a = jnp.dot(z, w1_ref[...], preferred_element_type=jnp.float32) + b1_ref[...]
    a = jnp.maximum(a, 0.0)
    z2 = jnp.dot(a, w2_ref[...], preferred_element_type=jnp.float32) + b2_ref[...]
    h2 = jnp.maximum(z2, 0.0)
    o_ref[...] = jnp.dot(h2, wr_ref[...], preferred_element_type=jnp.float32) + br_ref[...]


def _row_spec():
    return pl.BlockSpec((_ROW_BLK, D), lambda i: (i, 0))


def _part_spec():
    return pl.BlockSpec((2, _ROW_BLK, D), lambda i: (0, i, 0))


def _full_spec(shape):
    return pl.BlockSpec(shape, lambda i: tuple(0 for _ in shape))


def _mlp1(h, p, w1, b1, w2, b2):
    return pl.pallas_call(
        _mlp1_body,
        grid=(N_NODES // _ROW_BLK,),
        in_specs=[
            _row_spec(), _part_spec(),
            _full_spec((D, D)), _full_spec((1, D)),
            _full_spec((D, D)), _full_spec((1, D)),
        ],
        out_specs=_row_spec(),
        out_shape=jax.ShapeDtypeStruct((N_NODES, D), jnp.float32),
    )(h, p, w1, b1, w2, b2)


def _mlp2(h, p, w1, b1, w2, b2, wr, br):
    return pl.pallas_call(
        _mlp2_body,
        grid=(N_NODES // _ROW_BLK,),
        in_specs=[
            _row_spec(), _part_spec(),
            _full_spec((D, D)), _full_spec((1, D)),
            _full_spec((D, D)), _full_spec((1, D)),
            _full_spec((D, 1)), _full_spec((1, 1)),
        ],
        out_specs=pl.BlockSpec((_ROW_BLK, 1), lambda i: (i, 0)),
        out_shape=jax.ShapeDtypeStruct((N_NODES, 1), jnp.float32),
    )(h, p, w1, b1, w2, b2, wr, br)


@jax.jit
def kernel(x, edge_index, W1_0, b1_0, W2_0, b2_0, W1_1, b1_1, W2_1, b2_1, Wr, br):
    src = edge_index[0].astype(jnp.int32)
    dst = edge_index[1].astype(jnp.int32)
    trash = (N_NODES + (jnp.arange(PAD_PER_TILE, dtype=jnp.int32) % TRASH))
    src_g = jnp.concatenate(
        [src.reshape(NW, REAL_PER_TILE),
         jnp.zeros((NW, PAD_PER_TILE), jnp.int32)], axis=1,
    ).reshape(NC, NS, CPT, CH)
    dst_g = jnp.concatenate(
        [dst.reshape(NW, REAL_PER_TILE),
         jnp.broadcast_to(trash, (NW, PAD_PER_TILE))], axis=1,
    ).reshape(NC, NS, CPT, CH)

    b1_0r = b1_0.reshape(1, D)
    b2_0r = b2_0.reshape(1, D)
    b1_1r = b1_1.reshape(1, D)
    b2_1r = b2_1.reshape(1, D)
    brr = br.reshape(1, 1)

    agg = _make_agg_kernel()
    p0 = agg(x, src_g, dst_g)[:, :N_NODES, :]
    h1 = _mlp1(x, p0, W1_0, b1_0r, W2_0, b2_0r)
    p1 = agg(h1, src_g, dst_g)[:, :N_NODES, :]
    out = _mlp2(h1, p1, W1_1, b1_1r, W2_1, b2_1r, Wr, brr)
    return out


# speedup vs baseline: 2.3065x; 2.3065x over previous
import functools

import jax
import jax.numpy as jnp
from jax import lax
from jax.experimental import pallas as pl
from jax.experimental.pallas import tpu as pltpu
from jax.experimental.pallas import tpu_sc as plsc

N_NODES = 10000
N_EDGES = 320000
D = 128

NC = 2
NS = 16
NW = NC * NS
CH = 128
BL = 8
NBLK = 10
CPT = NBLK * BL
EDGES_PER_TILE = CPT * CH
REAL_PER_TILE = N_EDGES // NW
PAD_PER_TILE = EDGES_PER_TILE - REAL_PER_TILE
N_PAD = 10112
TRASH = N_PAD - N_NODES
ROWS_PER_TILE = N_PAD // NS


@functools.cache
def _make_agg_kernel():
    mesh = plsc.VectorSubcoreMesh(core_axis_name="c", subcore_axis_name="s")

    @functools.partial(
        pl.kernel,
        mesh=mesh,
        out_type=jax.ShapeDtypeStruct((NC, N_PAD, D), jnp.float32),
        scratch_types=[
            pltpu.VMEM((CPT, CH), jnp.int32),
            pltpu.VMEM((CPT, CH), jnp.int32),
            pltpu.VMEM((CH, D), jnp.float32),
            pltpu.VMEM_SHARED((N_PAD, D), jnp.float32),
            pltpu.SemaphoreType.DMA,
        ],
    )
    def agg(h_hbm, src_hbm, dst_hbm, out_hbm, src_v, dst_v, rows0, acc, sem0):
        c = lax.axis_index("c")
        s = lax.axis_index("s")

        pltpu.sync_copy(src_hbm.at[c, s], src_v)
        pltpu.sync_copy(dst_hbm.at[c, s], dst_v)

        def zrow(r, carry):
            for k in range(D // 16):
                rows0[r, pl.ds(k * 16, 16)] = jnp.zeros((16,), jnp.float32)
            return carry

        lax.fori_loop(0, CH, zrow, 0)
        base = s * ROWS_PER_TILE
        full = ROWS_PER_TILE // CH
        rem = ROWS_PER_TILE - full * CH
        for k in range(full):
            pltpu.sync_copy(rows0, acc.at[pl.ds(base + k * CH, CH)])
        if rem:
            pltpu.sync_copy(rows0.at[pl.ds(0, rem)],
                            acc.at[pl.ds(base + full * CH, rem)])
        plsc.subcore_barrier()

        def chunk(j, carry):
            pltpu.async_copy(h_hbm.at[src_v.at[j]], rows0, sem0).wait()
            pltpu.sync_copy(rows0, acc.at[dst_v.at[j]], add=True)
            return carry

        lax.fori_loop(0, CPT, chunk, 0)
        plsc.subcore_barrier()

        pltpu.sync_copy(acc.at[pl.ds(base, ROWS_PER_TILE)],
                        out_hbm.at[c, pl.ds(base, ROWS_PER_TILE)])

    return agg


_ROW_BLK = 1000


def _mlp1_body(h_ref, p_ref, w1_ref, b1_ref, w2_ref, b2_ref, o_ref):
    z = h_ref[...] + p_ref[0] + p_ref[1]
    a = jnp.dot(z, w1_ref[...], preferred_element_type=jnp.float32) + b1_ref[...]
    a = jnp.maximum(a, 0.0)
    z2 = jnp.dot(a, w2_ref[...], preferred_element_type=jnp.float32) + b2_ref[...]
    o_ref[...] = jnp.maximum(z2, 0.0)


def _mlp2_body(h_ref, p_ref, w1_ref, b1_ref, w2_ref, b2_ref, wr_ref, br_ref,
               o_ref):
    z = h_ref[...] + p_ref[0] + p_ref[1]
    a = jnp.dot(z, w1_ref[...], preferred_element_type=jnp.float32) + b1_ref[...]
    a = jnp.maximum(a, 0.0)
    z2 = jnp.dot(a, w2_ref[...], preferred_element_type=jnp.float32) + b2_ref[...]
    h2 = jnp.maximum(z2, 0.0)
    o_ref[...] = jnp.dot(h2, wr_ref[...], preferred_element_type=jnp.float32) + br_ref[...]


def _row_spec():
    return pl.BlockSpec((_ROW_BLK, D), lambda i: (i, 0))


def _part_spec():
    return pl.BlockSpec((2, _ROW_BLK, D), lambda i: (0, i, 0))


def _full_spec(shape):
    return pl.BlockSpec(shape, lambda i: tuple(0 for _ in shape))


def _mlp1(h, p, w1, b1, w2, b2):
    return pl.pallas_call(
        _mlp1_body,
        grid=(N_NODES // _ROW_BLK,),
        in_specs=[
            _row_spec(), _part_spec(),
            _full_spec((D, D)), _full_spec((1, D)),
            _full_spec((D, D)), _full_spec((1, D)),
        ],
        out_specs=_row_spec(),
        out_shape=jax.ShapeDtypeStruct((N_NODES, D), jnp.float32),
    )(h, p, w1, b1, w2, b2)


def _mlp2(h, p, w1, b1, w2, b2, wr, br):
    return pl.pallas_call(
        _mlp2_body,
        grid=(N_NODES // _ROW_BLK,),
        in_specs=[
            _row_spec(), _part_spec(),
            _full_spec((D, D)), _full_spec((1, D)),
            _full_spec((D, D)), _full_spec((1, D)),
            _full_spec((D, 1)), _full_spec((1, 1)),
        ],
        out_specs=pl.BlockSpec((_ROW_BLK, 1), lambda i: (i, 0)),
        out_shape=jax.ShapeDtypeStruct((N_NODES, 1), jnp.float32),
    )(h, p, w1, b1, w2, b2, wr, br)


@jax.jit
def kernel(x, edge_index, W1_0, b1_0, W2_0, b2_0, W1_1, b1_1, W2_1, b2_1, Wr, br):
    src = edge_index[0].astype(jnp.int32)
    dst = edge_index[1].astype(jnp.int32)
    pad_src = (jnp.arange(PAD_PER_TILE, dtype=jnp.int32) * 41) % N_NODES
    w_ids = jnp.arange(NW, dtype=jnp.int32) % NS
    pad_dst = (N_NODES + 7 * w_ids[:, None]
               + (jnp.arange(PAD_PER_TILE, dtype=jnp.int32)[None, :] % 7))
    src_g = jnp.concatenate(
        [src.reshape(NW, REAL_PER_TILE),
         jnp.broadcast_to(pad_src, (NW, PAD_PER_TILE))], axis=1,
    ).reshape(NC, NS, CPT, CH)
    dst_g = jnp.concatenate(
        [dst.reshape(NW, REAL_PER_TILE), pad_dst], axis=1,
    ).reshape(NC, NS, CPT, CH)

    b1_0r = b1_0.reshape(1, D)
    b2_0r = b2_0.reshape(1, D)
    b1_1r = b1_1.reshape(1, D)
    b2_1r = b2_1.reshape(1, D)
    brr = br.reshape(1, 1)

    agg = _make_agg_kernel()
    p0 = agg(x, src_g, dst_g)[:, :N_NODES, :]
    h1 = _mlp1(x, p0, W1_0, b1_0r, W2_0, b2_0r)
    p1 = agg(h1, src_g, dst_g)[:, :N_NODES, :]
    out = _mlp2(h1, p1, W1_1, b1_1r, W2_1, b2_1r, Wr, brr)
    return out


# speedup vs baseline: 3.0449x; 1.3201x over previous
import functools

import jax
import jax.numpy as jnp
from jax import lax
from jax.experimental import pallas as pl
from jax.experimental.pallas import tpu as pltpu
from jax.experimental.pallas import tpu_sc as plsc

N_NODES = 10000
N_EDGES = 320000
D = 128

NC = 2
NS = 16
NW = NC * NS
CH = 128
BL = 8
NBLK = 10
CPT = NBLK * BL
EDGES_PER_TILE = CPT * CH
REAL_PER_TILE = N_EDGES // NW
PAD_PER_TILE = EDGES_PER_TILE - REAL_PER_TILE
N_PAD = 10112
TRASH = N_PAD - N_NODES
ROWS_PER_TILE = N_PAD // NS


@functools.cache
def _make_agg_kernel():
    mesh = plsc.VectorSubcoreMesh(core_axis_name="c", subcore_axis_name="s")

    @functools.partial(
        pl.kernel,
        mesh=mesh,
        out_type=jax.ShapeDtypeStruct((NC, N_PAD, D), jnp.float32),
        scratch_types=[
            pltpu.VMEM((2, BL, CH), jnp.int32),
            pltpu.VMEM((CH, D), jnp.float32),
            pltpu.VMEM((CH, D), jnp.float32),
            pltpu.VMEM_SHARED((N_PAD, D), jnp.float32),
            pltpu.SemaphoreType.DMA,
            pltpu.SemaphoreType.DMA,
        ],
    )
    def agg(h_hbm, idx_hbm, out_hbm, ib, rows0, rows1, acc, sem0, sem1):
        c = lax.axis_index("c")
        s = lax.axis_index("s")
        rows = (rows0, rows1)
        sems = (sem0, sem1)

        def zrow(r, carry):
            for k in range(D // 16):
                rows0[r, pl.ds(k * 16, 16)] = jnp.zeros((16,), jnp.float32)
            return carry

        lax.fori_loop(0, CH, zrow, 0)
        base = s * ROWS_PER_TILE
        full = ROWS_PER_TILE // CH
        rem = ROWS_PER_TILE - full * CH
        for k in range(full):
            pltpu.sync_copy(rows0, acc.at[pl.ds(base + k * CH, CH)])
        if rem:
            pltpu.sync_copy(rows0.at[pl.ds(0, rem)],
                            acc.at[pl.ds(base + full * CH, rem)])
        plsc.subcore_barrier()

        def block(b, carry):
            pltpu.sync_copy(idx_hbm.at[c, s, b], ib)
            pltpu.async_copy(h_hbm.at[ib.at[0, 0]], rows0, sem0)
            pltpu.async_copy(h_hbm.at[ib.at[0, 1]], rows1, sem1)
            for k in range(BL):
                r, sem = rows[k % 2], sems[k % 2]
                pltpu.make_async_copy(h_hbm.at[ib.at[0, k]], r, sem).wait()
                pltpu.sync_copy(r, acc.at[ib.at[1, k]], add=True)
                if k + 2 < BL:
                    pltpu.async_copy(h_hbm.at[ib.at[0, k + 2]], r, sem)
            return carry

        lax.fori_loop(0, NBLK, block, 0)
        plsc.subcore_barrier()

        pltpu.sync_copy(acc.at[pl.ds(base, ROWS_PER_TILE)],
                        out_hbm.at[c, pl.ds(base, ROWS_PER_TILE)])

    return agg


_ROW_BLK = 1000


def _mlp1_body(h_ref, p_ref, w1_ref, b1_ref, w2_ref, b2_ref, o_ref):
    z = h_ref[...] + p_ref[0] + p_ref[1]
    a = jnp.dot(z, w1_ref[...], preferred_element_type=jnp.float32) + b1_ref[...]
    a = jnp.maximum(a, 0.0)
    z2 = jnp.dot(a, w2_ref[...], preferred_element_type=jnp.float32) + b2_ref[...]
    o_ref[...] = jnp.maximum(z2, 0.0)


def _mlp2_body(h_ref, p_ref, w1_ref, b1_ref, w2_ref, b2_ref, wr_ref, br_ref,
               o_ref):
    z = h_ref[...] + p_ref[0] + p_ref[1]
    a = jnp.dot(z, w1_ref[...], preferred_element_type=jnp.float32) + b1_ref[...]
    a = jnp.maximum(a, 0.0)
    z2 = jnp.dot(a, w2_ref[...], preferred_element_type=jnp.float32) + b2_ref[...]
    h2 = jnp.maximum(z2, 0.0)
    o_ref[...] = jnp.dot(h2, wr_ref[...], preferred_element_type=jnp.float32) + br_ref[...]


def _row_spec():
    return pl.BlockSpec((_ROW_BLK, D), lambda i: (i, 0))


def _part_spec():
    return pl.BlockSpec((2, _ROW_BLK, D), lambda i: (0, i, 0))


def _full_spec(shape):
    return pl.BlockSpec(shape, lambda i: tuple(0 for _ in shape))


def _mlp1(h, p, w1, b1, w2, b2):
    return pl.pallas_call(
        _mlp1_body,
        grid=(N_NODES // _ROW_BLK,),
        in_specs=[
            _row_spec(), _part_spec(),
            _full_spec((D, D)), _full_spec((1, D)),
            _full_spec((D, D)), _full_spec((1, D)),
        ],
        out_specs=_row_spec(),
        out_shape=jax.ShapeDtypeStruct((N_NODES, D), jnp.float32),
    )(h, p, w1, b1, w2, b2)


def _mlp2(h, p, w1, b1, w2, b2, wr, br):
    return pl.pallas_call(
        _mlp2_body,
        grid=(N_NODES // _ROW_BLK,),
        in_specs=[
            _row_spec(), _part_spec(),
            _full_spec((D, D)), _full_spec((1, D)),
            _full_spec((D, D)), _full_spec((1, D)),
            _full_spec((D, 1)), _full_spec((1, 1)),
        ],
        out_specs=pl.BlockSpec((_ROW_BLK, 1), lambda i: (i, 0)),
        out_shape=jax.ShapeDtypeStruct((N_NODES, 1), jnp.float32),
    )(h, p, w1, b1, w2, b2, wr, br)


@jax.jit
def kernel(x, edge_index, W1_0, b1_0, W2_0, b2_0, W1_1, b1_1, W2_1, b2_1, Wr, br):
    src = edge_index[0].astype(jnp.int32)
    dst = edge_index[1].astype(jnp.int32)
    pad_src = (jnp.arange(PAD_PER_TILE, dtype=jnp.int32) * 41) % N_NODES
    w_ids = jnp.arange(NW, dtype=jnp.int32) % NS
    pad_dst = (N_NODES + 7 * w_ids[:, None]
               + (jnp.arange(PAD_PER_TILE, dtype=jnp.int32)[None, :] % 7))
    src_t = jnp.concatenate(
        [src.reshape(NW, REAL_PER_TILE),
         jnp.broadcast_to(pad_src, (NW, PAD_PER_TILE))], axis=1,
    ).reshape(NW, NBLK, BL, CH)
    dst_t = jnp.concatenate(
        [dst.reshape(NW, REAL_PER_TILE), pad_dst], axis=1,
    ).reshape(NW, NBLK, BL, CH)
    idx_g = jnp.stack([src_t, dst_t], axis=2).reshape(NC, NS, NBLK, 2, BL, CH)

    b1_0r = b1_0.reshape(1, D)
    b2_0r = b2_0.reshape(1, D)
    b1_1r = b1_1.reshape(1, D)
    b2_1r = b2_1.reshape(1, D)
    brr = br.reshape(1, 1)

    agg = _make_agg_kernel()
    p0 = agg(x, idx_g)[:, :N_NODES, :]
    h1 = _mlp1(x, p0, W1_0, b1_0r, W2_0, b2_0r)
    p1 = agg(h1, idx_g)[:, :N_NODES, :]
    out = _mlp2(h1, p1, W1_1, b1_1r, W2_1, b2_1r, Wr, brr)
    return out


# speedup vs baseline: 3.3751x; 1.1084x over previous
import functools

import jax
import jax.numpy as jnp
from jax import lax
from jax.experimental import pallas as pl
from jax.experimental.pallas import tpu as pltpu
from jax.experimental.pallas import tpu_sc as plsc

N_NODES = 10000
N_EDGES = 320000
D = 128

NC = 2
NS = 16
NW = NC * NS
CH = 128
BL = 8
NBLK = 10
CPT = NBLK * BL
EDGES_PER_TILE = CPT * CH
REAL_PER_TILE = N_EDGES // NW
PAD_PER_TILE = EDGES_PER_TILE - REAL_PER_TILE
N_PAD = 10112
TRASH = N_PAD - N_NODES
ROWS_PER_TILE = N_PAD // NS


@functools.cache
def _make_agg_kernel():
    mesh = plsc.VectorSubcoreMesh(core_axis_name="c", subcore_axis_name="s")

    @functools.partial(
        pl.kernel,
        mesh=mesh,
        out_type=jax.ShapeDtypeStruct((NC, N_PAD, D), jnp.float32),
        scratch_types=[
            pltpu.VMEM((2, BL, CH), jnp.int32),
            pltpu.VMEM((2, BL, CH), jnp.int32),
            pltpu.VMEM((CH, D), jnp.float32),
            pltpu.VMEM((CH, D), jnp.float32),
            pltpu.VMEM_SHARED((N_PAD, D), jnp.float32),
            pltpu.SemaphoreType.DMA,
            pltpu.SemaphoreType.DMA,
            pltpu.SemaphoreType.DMA,
            pltpu.SemaphoreType.DMA,
        ],
    )
    def agg(h_hbm, idx_hbm, out_hbm, ib0, ib1, rows0, rows1, acc,
            semi0, semi1, sem0, sem1):
        c = lax.axis_index("c")
        s = lax.axis_index("s")
        rows = (rows0, rows1)
        sems = (sem0, sem1)

        def zrow(r, carry):
            for k in range(D // 16):
                rows0[r, pl.ds(k * 16, 16)] = jnp.zeros((16,), jnp.float32)
            return carry

        lax.fori_loop(0, CH, zrow, 0)
        base = s * ROWS_PER_TILE
        full = ROWS_PER_TILE // CH
        rem = ROWS_PER_TILE - full * CH
        for k in range(full):
            pltpu.sync_copy(rows0, acc.at[pl.ds(base + k * CH, CH)])
        if rem:
            pltpu.sync_copy(rows0.at[pl.ds(0, rem)],
                            acc.at[pl.ds(base + full * CH, rem)])
        plsc.subcore_barrier()

        pltpu.sync_copy(idx_hbm.at[c, s, 0], ib0)
        pltpu.async_copy(idx_hbm.at[c, s, 1], ib1, semi1)
        pltpu.async_copy(h_hbm.at[ib0.at[0, 0]], rows0, sem0)
        pltpu.async_copy(h_hbm.at[ib0.at[0, 1]], rows1, sem1)

        def pair(p, carry):
            bnext0 = jnp.minimum(2 * p + 2, NBLK - 1)
            bnext1 = jnp.minimum(2 * p + 3, NBLK - 1)
            for m in range(2 * BL):
                if m == 6:
                    pltpu.make_async_copy(
                        idx_hbm.at[c, s, 0], ib1, semi1).wait()
                if m == 8:
                    pltpu.async_copy(idx_hbm.at[c, s, bnext0], ib0, semi0)
                if m == 14:
                    pltpu.make_async_copy(
                        idx_hbm.at[c, s, 0], ib0, semi0).wait()
                r, sem = rows[m % 2], sems[m % 2]
                ib_cur, k_cur = (ib0, m) if m < BL else (ib1, m - BL)
                pltpu.make_async_copy(h_hbm.at[ib_cur.at[0, k_cur]], r,
                                      sem).wait()
                pltpu.sync_copy(r, acc.at[ib_cur.at[1, k_cur]], add=True)
                m2 = m + 2
                ib_n, k_n = ((ib0, m2) if m2 < BL else
                             (ib1, m2 - BL) if m2 < 2 * BL else
                             (ib0, m2 - 2 * BL))
                pltpu.async_copy(h_hbm.at[ib_n.at[0, k_n]], r, sem)
            pltpu.async_copy(idx_hbm.at[c, s, bnext1], ib1, semi1)
            return carry

        lax.fori_loop(0, NBLK // 2, pair, 0)
        pltpu.make_async_copy(h_hbm.at[ib0.at[0, 0]], rows0, sem0).wait()
        pltpu.make_async_copy(h_hbm.at[ib0.at[0, 1]], rows1, sem1).wait()
        pltpu.make_async_copy(idx_hbm.at[c, s, 0], ib1, semi1).wait()
        plsc.subcore_barrier()

        pltpu.sync_copy(acc.at[pl.ds(base, ROWS_PER_TILE)],
                        out_hbm.at[c, pl.ds(base, ROWS_PER_TILE)])

    return agg


_ROW_BLK = 1000


def _mlp1_body(h_ref, p_ref, w1_ref, b1_ref, w2_ref, b2_ref, o_ref):
    z = h_ref[...] + p_ref[0] + p_ref[1]
    a = jnp.dot(z, w1_ref[...], preferred_element_type=jnp.float32) + b1_ref[...]
    a = jnp.maximum(a, 0.0)
    z2 = jnp.dot(a, w2_ref[...], preferred_element_type=jnp.float32) + b2_ref[...]
    o_ref[...] = jnp.maximum(z2, 0.0)


def _mlp2_body(h_ref, p_ref, w1_ref, b1_ref, w2_ref, b2_ref, wr_ref, br_ref,
               o_ref):
    z = h_ref[...] + p_ref[0] + p_ref[1]
    a = jnp.dot(z, w1_ref[...], preferred_element_type=jnp.float32) + b1_ref[...]
    a = jnp.maximum(a, 0.0)
    z2 = jnp.dot(a, w2_ref[...], preferred_element_type=jnp.float32) + b2_ref[...]
    h2 = jnp.maximum(z2, 0.0)
    o_ref[...] = jnp.dot(h2, wr_ref[...], preferred_element_type=jnp.float32) + br_ref[...]


def _row_spec():
    return pl.BlockSpec((_ROW_BLK, D), lambda i: (i, 0))


def _part_spec():
    return pl.BlockSpec((2, _ROW_BLK, D), lambda i: (0, i, 0))


def _full_spec(shape):
    return pl.BlockSpec(shape, lambda i: tuple(0 for _ in shape))


def _mlp1(h, p, w1, b1, w2, b2):
    return pl.pallas_call(
        _mlp1_body,
        grid=(N_NODES // _ROW_BLK,),
        in_specs=[
            _row_spec(), _part_spec(),
            _full_spec((D, D)), _full_spec((1, D)),
            _full_spec((D, D)), _full_spec((1, D)),
        ],
        out_specs=_row_spec(),
        out_shape=jax.ShapeDtypeStruct((N_NODES, D), jnp.float32),
    )(h, p, w1, b1, w2, b2)


def _mlp2(h, p, w1, b1, w2, b2, wr, br):
    return pl.pallas_call(
        _mlp2_body,
        grid=(N_NODES // _ROW_BLK,),
        in_specs=[
            _row_spec(), _part_spec(),
            _full_spec((D, D)), _full_spec((1, D)),
            _full_spec((D, D)), _full_spec((1, D)),
            _full_spec((D, 1)), _full_spec((1, 1)),
        ],
        out_specs=pl.BlockSpec((_ROW_BLK, 1), lambda i: (i, 0)),
        out_shape=jax.ShapeDtypeStruct((N_NODES, 1), jnp.float32),
    )(h, p, w1, b1, w2, b2, wr, br)


@jax.jit
def kernel(x, edge_index, W1_0, b1_0, W2_0, b2_0, W1_1, b1_1, W2_1, b2_1, Wr, br):
    src = edge_index[0].astype(jnp.int32)
    dst = edge_index[1].astype(jnp.int32)
    pad_src = (jnp.arange(PAD_PER_TILE, dtype=jnp.int32) * 41) % N_NODES
    w_ids = jnp.arange(NW, dtype=jnp.int32) % NS
    pad_dst = (N_NODES + 7 * w_ids[:, None]
               + (jnp.arange(PAD_PER_TILE, dtype=jnp.int32)[None, :] % 7))
    src_t = jnp.concatenate(
        [src.reshape(NW, REAL_PER_TILE),
         jnp.broadcast_to(pad_src, (NW, PAD_PER_TILE))], axis=1,
    ).reshape(NW, NBLK, BL, CH)
    dst_t = jnp.concatenate(
        [dst.reshape(NW, REAL_PER_TILE), pad_dst], axis=1,
    ).reshape(NW, NBLK, BL, CH)
    idx_g = jnp.stack([src_t, dst_t], axis=2).reshape(NC, NS, NBLK, 2, BL, CH)

    b1_0r = b1_0.reshape(1, D)
    b2_0r = b2_0.reshape(1, D)
    b1_1r = b1_1.reshape(1, D)
    b2_1r = b2_1.reshape(1, D)
    brr = br.reshape(1, 1)

    agg = _make_agg_kernel()
    p0 = agg(x, idx_g)[:, :N_NODES, :]
    h1 = _mlp1(x, p0, W1_0, b1_0r, W2_0, b2_0r)
    p1 = agg(h1, idx_g)[:, :N_NODES, :]
    out = _mlp2(h1, p1, W1_1, b1_1r, W2_1, b2_1r, Wr, brr)
    return out


# speedup vs baseline: 3.5420x; 1.0495x over previous
import functools

import jax
import jax.numpy as jnp
from jax import lax
from jax.experimental import pallas as pl
from jax.experimental.pallas import tpu as pltpu
from jax.experimental.pallas import tpu_sc as plsc

N_NODES = 10000
N_EDGES = 320000
D = 128

NC = 2
NS = 16
NW = NC * NS
CH = 128
BL = 8
NBLK = 10
CPT = NBLK * BL
EDGES_PER_TILE = CPT * CH
REAL_PER_TILE = N_EDGES // NW
PAD_PER_TILE = EDGES_PER_TILE - REAL_PER_TILE
N_PAD = 10112
TRASH = N_PAD - N_NODES
ROWS_PER_TILE = N_PAD // NS


@functools.cache
def _make_agg_kernel():
    mesh = plsc.VectorSubcoreMesh(core_axis_name="c", subcore_axis_name="s")

    @functools.partial(
        pl.kernel,
        mesh=mesh,
        out_type=jax.ShapeDtypeStruct((NC, N_PAD, D), jnp.float32),
        scratch_types=[
            pltpu.VMEM((2, BL, CH), jnp.int32),
            pltpu.VMEM((2, BL, CH), jnp.int32),
            pltpu.VMEM((CH, D), jnp.float32),
            pltpu.VMEM((CH, D), jnp.float32),
            pltpu.VMEM_SHARED((N_PAD, D), jnp.float32),
            pltpu.SemaphoreType.DMA,
            pltpu.SemaphoreType.DMA,
            pltpu.SemaphoreType.DMA,
            pltpu.SemaphoreType.DMA,
        ],
    )
    def agg(h_hbm, idx_hbm, out_hbm, ib0, ib1, rows0, rows1, acc,
            semi0, semi1, sem0, sem1):
        c = lax.axis_index("c")
        s = lax.axis_index("s")
        rows = (rows0, rows1)
        sems = (sem0, sem1)

        def zrow(r, carry):
            for k in range(D // 16):
                rows0[r, pl.ds(k * 16, 16)] = jnp.zeros((16,), jnp.float32)
            return carry

        lax.fori_loop(0, CH, zrow, 0)
        base = s * ROWS_PER_TILE
        full = ROWS_PER_TILE // CH
        rem = ROWS_PER_TILE - full * CH
        for k in range(full):
            pltpu.sync_copy(rows0, acc.at[pl.ds(base + k * CH, CH)])
        if rem:
            pltpu.sync_copy(rows0.at[pl.ds(0, rem)],
                            acc.at[pl.ds(base + full * CH, rem)])
        plsc.subcore_barrier()

        pltpu.sync_copy(idx_hbm.at[c, s, 0], ib0)
        pltpu.async_copy(idx_hbm.at[c, s, 1], ib1, semi1)
        pltpu.async_copy(h_hbm.at[ib0.at[0, 0]], rows0, sem0)
        pltpu.async_copy(h_hbm.at[ib0.at[0, 1]], rows1, sem1)

        def pair(p, carry):
            bnext0 = jnp.minimum(2 * p + 2, NBLK - 1)
            bnext1 = jnp.minimum(2 * p + 3, NBLK - 1)
            for m in range(2 * BL):
                if m == 6:
                    pltpu.make_async_copy(
                        idx_hbm.at[c, s, 0], ib1, semi1).wait()
                if m == 8:
                    pltpu.async_copy(idx_hbm.at[c, s, bnext0], ib0, semi0)
                if m == 14:
                    pltpu.make_async_copy(
                        idx_hbm.at[c, s, 0], ib0, semi0).wait()
                r, sem = rows[m % 2], sems[m % 2]
                ib_cur, k_cur = (ib0, m) if m < BL else (ib1, m - BL)
                pltpu.make_async_copy(h_hbm.at[ib_cur.at[0, k_cur]], r,
                                      sem).wait()
                pltpu.sync_copy(r, acc.at[ib_cur.at[1, k_cur]], add=True)
                m2 = m + 2
                ib_n, k_n = ((ib0, m2) if m2 < BL else
                             (ib1, m2 - BL) if m2 < 2 * BL else
                             (ib0, m2 - 2 * BL))
                pltpu.async_copy(h_hbm.at[ib_n.at[0, k_n]], r, sem)
            pltpu.async_copy(idx_hbm.at[c, s, bnext1], ib1, semi1)
            return carry

        lax.fori_loop(0, NBLK // 2, pair, 0)
        pltpu.make_async_copy(h_hbm.at[ib0.at[0, 0]], rows0, sem0).wait()
        pltpu.make_async_copy(h_hbm.at[ib0.at[0, 1]], rows1, sem1).wait()
        pltpu.make_async_copy(idx_hbm.at[c, s, 0], ib1, semi1).wait()
        plsc.subcore_barrier()

        pltpu.sync_copy(acc.at[pl.ds(base, ROWS_PER_TILE)],
                        out_hbm.at[c, pl.ds(base, ROWS_PER_TILE)])

    return agg


_ROW_BLK = 1000


def _mlp1_body(h_ref, p_ref, w1_ref, b1_ref, w2_ref, b2_ref, o_ref):
    z = h_ref[...] + p_ref[0] + p_ref[1]
    a = jnp.dot(z, w1_ref[...], preferred_element_type=jnp.float32) + b1_ref[...]
    a = jnp.maximum(a, 0.0)
    z2 = jnp.dot(a, w2_ref[...], preferred_element_type=jnp.float32) + b2_ref[...]
    o_ref[...] = jnp.maximum(z2, 0.0)


def _mlp2_body(h_ref, p_ref, w1_ref, b1_ref, w2_ref, b2_ref, wr_ref, br_ref,
               o_ref):
    z = h_ref[...] + p_ref[0] + p_ref[1]
    a = jnp.dot(z, w1_ref[...], preferred_element_type=jnp.float32) + b1_ref[...]
    a = jnp.maximum(a, 0.0)
    z2 = jnp.dot(a, w2_ref[...], preferred_element_type=jnp.float32) + b2_ref[...]
    h2 = jnp.maximum(z2, 0.0)
    o_ref[...] = jnp.dot(h2, wr_ref[...], preferred_element_type=jnp.float32) + br_ref[...]


def _row_spec():
    return pl.BlockSpec((_ROW_BLK, D), lambda i: (i, 0))


def _part_spec():
    return pl.BlockSpec((2, _ROW_BLK, D), lambda i: (0, i, 0))


def _full_spec(shape):
    return pl.BlockSpec(shape, lambda i: tuple(0 for _ in shape))


def _mlp1(h, p, w1, b1, w2, b2):
    return pl.pallas_call(
        _mlp1_body,
        grid=(N_NODES // _ROW_BLK,),
        in_specs=[
            _row_spec(), _part_spec(),
            _full_spec((D, D)), _full_spec((1, D)),
            _full_spec((D, D)), _full_spec((1, D)),
        ],
        out_specs=_row_spec(),
        out_shape=jax.ShapeDtypeStruct((N_NODES, D), jnp.float32),
    )(h, p, w1, b1, w2, b2)


def _cast_and_pack_edges(edge_index):
    src = edge_index[0].astype(jnp.int32)
    dst = edge_index[1].astype(jnp.int32)
    pad_src = (jnp.arange(PAD_PER_TILE, dtype=jnp.int32) * 41) % N_NODES
    w_ids = jnp.arange(NW, dtype=jnp.int32) % NS
    pad_dst = (N_NODES + 7 * w_ids[:, None]
               + (jnp.arange(PAD_PER_TILE, dtype=jnp.int32)[None, :] % 7))
    src_t = jnp.concatenate(
        [src.reshape(NW, REAL_PER_TILE),
         jnp.broadcast_to(pad_src, (NW, PAD_PER_TILE))], axis=1,
    ).reshape(NW, NBLK, BL, CH)
    dst_t = jnp.concatenate(
        [dst.reshape(NW, REAL_PER_TILE), pad_dst], axis=1,
    ).reshape(NW, NBLK, BL, CH)
    return jnp.stack([src_t, dst_t], axis=2).reshape(NC, NS, NBLK, 2, BL, CH)


def _mlp2(h, p, w1, b1, w2, b2, wr, br):
    return pl.pallas_call(
        _mlp2_body,
        grid=(N_NODES // _ROW_BLK,),
        in_specs=[
            _row_spec(), _part_spec(),
            _full_spec((D, D)), _full_spec((1, D)),
            _full_spec((D, D)), _full_spec((1, D)),
            _full_spec((D, 1)), _full_spec((1, 1)),
        ],
        out_specs=pl.BlockSpec((_ROW_BLK, 1), lambda i: (i, 0)),
        out_shape=jax.ShapeDtypeStruct((N_NODES, 1), jnp.float32),
    )(h, p, w1, b1, w2, b2, wr, br)


@jax.jit
def kernel(x, edge_index, W1_0, b1_0, W2_0, b2_0, W1_1, b1_1, W2_1, b2_1, Wr, br):
    idx_g = _cast_and_pack_edges(edge_index)

    b1_0r = b1_0.reshape(1, D)
    b2_0r = b2_0.reshape(1, D)
    b1_1r = b1_1.reshape(1, D)
    b2_1r = b2_1.reshape(1, D)
    brr = br.reshape(1, 1)

    agg = _make_agg_kernel()
    p0 = agg(x, idx_g)
    h1 = _mlp1(x, p0, W1_0, b1_0r, W2_0, b2_0r)
    p1 = agg(h1, idx_g)
    out = _mlp2(h1, p1, W1_1, b1_1r, W2_1, b2_1r, Wr, brr)
    return out
